# baseline (speedup 1.0000x reference)
import functools

import jax
import jax.numpy as jnp
from jax import lax
from jax.experimental import pallas as pl
from jax.experimental.pallas import tpu as pltpu
from jax.experimental.pallas import tpu_sc as plsc

F32 = jnp.float32
I32 = jnp.int32

NC = 2
NS = 16
LANES = 16

_HIGHEST = lax.Precision.HIGHEST


def _iota16():
    return lax.iota(I32, LANES)


def _lane_bcast(v, idx_const):
    dnums = lax.GatherDimensionNumbers(
        offset_dims=(), collapsed_slice_dims=(0,), start_index_map=(0,))
    return lax.gather(v, idx_const[:, None], dnums, slice_sizes=(1,),
                      mode=lax.GatherScatterMode.PROMISE_IN_BOUNDS)


def _segadd4(updf, idx, *, nseg_pad, ch):
    npts = idx.shape[0]
    tpw = npts // (NC * NS)
    nch = tpw // ch
    se = nseg_pad * 4 // NS
    ce = ch * 4

    mesh = plsc.VectorSubcoreMesh(core_axis_name="c", subcore_axis_name="s",
                                  num_cores=NC, num_subcores=NS)

    @functools.partial(
        pl.kernel, mesh=mesh,
        out_type=jax.ShapeDtypeStruct((NC * nseg_pad * 4,), F32),
        scratch_types=[
            pltpu.VMEM_SHARED((nseg_pad * 4,), F32),
            pltpu.VMEM((ch + LANES,), I32),
            pltpu.VMEM((ce,), I32),
            pltpu.VMEM((ce,), F32),
            pltpu.VMEM((se,), F32),
        ],
    )
    def k(upd_hbm, idx_hbm, out_hbm, acc, idxb, eidx, updb, zbuf):
        c = lax.axis_index("c")
        t = lax.axis_index("s")
        w = t * NC + c
        it = _iota16()
        f4 = it & 3
        z16 = it.astype(F32) * 0.0

        def zv(i, _):
            zbuf[pl.ds(pl.multiple_of(i * LANES, LANES), LANES)] = z16
            return _

        lax.fori_loop(0, se // LANES, zv, 0)
        toff = pl.multiple_of(t * se, se)
        pltpu.sync_copy(zbuf, acc.at[pl.ds(toff, se)])
        plsc.subcore_barrier()

        def chunk(g, _):
            off = pl.multiple_of(w * tpw + g * ch, ch)
            pltpu.sync_copy(idx_hbm.at[pl.ds(off, ch)], idxb.at[pl.ds(0, ch)])
            eoff = pl.multiple_of(off * 4, ce)
            pltpu.sync_copy(upd_hbm.at[pl.ds(eoff, ce)], updb)

            def build16(j, _2):
                base16 = idxb[pl.ds(pl.multiple_of(j * 4, 4), LANES)]
                ids4 = _lane_bcast(base16, it >> 2)
                eidx[pl.ds(pl.multiple_of(j * LANES, LANES), LANES)] = (
                    ids4 * 4 + f4)
                return _2

            lax.fori_loop(0, ce // LANES, build16, 0)
            pltpu.sync_copy(updb, acc.at[eidx], add=True)
            return _

        lax.fori_loop(0, nch, chunk, 0)
        plsc.subcore_barrier()
        ooff = pl.multiple_of(c * nseg_pad * 4 + t * se, se)
        pltpu.sync_copy(acc.at[pl.ds(toff, se)], out_hbm.at[pl.ds(ooff, se)])

    return k(updf, idx)


def _norm4(sums2_flat):
    ne2 = sums2_flat.shape[0]
    ne = ne2 // NC
    per_w = ne // (NC * NS)
    nvec = per_w // LANES

    mesh = plsc.VectorSubcoreMesh(core_axis_name="c", subcore_axis_name="s",
                                  num_cores=NC, num_subcores=NS)

    @functools.partial(
        pl.kernel, mesh=mesh,
        out_type=jax.ShapeDtypeStruct((ne,), F32),
        scratch_types=[pltpu.VMEM((per_w,), F32), pltpu.VMEM((per_w,), F32)],
    )
    def k(sums_hbm, out_hbm, buf, buf2):
        w = lax.axis_index("s") * NC + lax.axis_index("c")
        toff = pl.multiple_of(w * per_w, per_w)
        toff2 = pl.multiple_of(ne + w * per_w, per_w)
        pltpu.sync_copy(sums_hbm.at[pl.ds(toff, per_w)], buf)
        pltpu.sync_copy(sums_hbm.at[pl.ds(toff2, per_w)], buf2)
        it = _iota16()
        p3 = it | 3

        def body(i, _):
            s = pl.ds(i * LANES, LANES)
            g = buf[s] + buf2[s]
            cn = jnp.maximum(_lane_bcast(g, p3), 1.0)
            r = g / cn
            mf = (((it & 3) - 3) >> 31).astype(F32) + 1.0
            buf[s] = r * (1.0 - mf) + mf
            return _

        lax.fori_loop(0, nvec, body, 0)
        pltpu.sync_copy(buf, out_hbm.at[pl.ds(toff, per_w)])

    return k(sums2_flat)


def _gather_sub4(table_flat, idx, base_flat, *, cp, nch):
    npts = idx.shape[0]
    assert npts == NC * NS * nch * cp
    ce = cp * 4

    mesh = plsc.VectorSubcoreMesh(core_axis_name="c", subcore_axis_name="s",
                                  num_cores=NC, num_subcores=NS)

    @functools.partial(
        pl.kernel, mesh=mesh,
        out_type=jax.ShapeDtypeStruct((npts * 4,), F32),
        scratch_types=[
            pltpu.VMEM((cp + LANES,), I32),
            pltpu.VMEM((ce,), I32),
            pltpu.VMEM((ce,), F32),
            pltpu.VMEM((ce,), F32),
            pltpu.SemaphoreType.DMA,
        ],
    )
    def k(tab_hbm, idx_hbm, base_hbm, out_hbm, idxb, eidx, g, b, sem):
        w = lax.axis_index("s") * NC + lax.axis_index("c")
        it = _iota16()
        f4 = it & 3

        def chunk(ci, _):
            off = pl.multiple_of((w * nch + ci) * cp, cp)
            pltpu.sync_copy(idx_hbm.at[pl.ds(off, cp)], idxb.at[pl.ds(0, cp)])

            def build16(j, _):
                base16 = idxb[pl.ds(j * 4, LANES)]
                ids4 = _lane_bcast(base16, it >> 2)
                eidx[pl.ds(j * LANES, LANES)] = ids4 * 4 + f4
                return _

            lax.fori_loop(0, ce // LANES, build16, 0)
            pltpu.async_copy(tab_hbm.at[eidx], g, sem).wait()
            eoff = pl.multiple_of(off * 4, ce)
            pltpu.sync_copy(base_hbm.at[pl.ds(eoff, ce)], b)

            def sub(j, _):
                s = pl.ds(j * LANES, LANES)
                b[s] = b[s] - g[s]
                return _

            lax.fori_loop(0, ce // LANES, sub, 0)
            pltpu.sync_copy(b, out_hbm.at[pl.ds(eoff, ce)])
            return _

        lax.fori_loop(0, nch, chunk, 0)

    return k(table_flat, idx, base_flat)


def _gather_rows(table, idx, *, cp, nch):
    npts = idx.shape[0]
    assert npts == NC * NS * nch * cp
    d = table.shape[1]

    mesh = plsc.VectorSubcoreMesh(core_axis_name="c", subcore_axis_name="s",
                                  num_cores=NC, num_subcores=NS)

    @functools.partial(
        pl.kernel, mesh=mesh,
        out_type=jax.ShapeDtypeStruct((npts, d), F32),
        scratch_types=[
            pltpu.VMEM((cp,), I32),
            pltpu.VMEM((cp, d), F32),
            pltpu.SemaphoreType.DMA,
        ],
    )
    def k(tab_hbm, idx_hbm, out_hbm, idxb, rows, sem):
        w = lax.axis_index("s") * NC + lax.axis_index("c")

        def chunk(ci, _):
            off = pl.multiple_of((w * nch + ci) * cp, cp)
            pltpu.sync_copy(idx_hbm.at[pl.ds(off, cp)], idxb)
            pltpu.async_copy(tab_hbm.at[idxb], rows, sem).wait()
            pltpu.sync_copy(rows, out_hbm.at[pl.ds(off, cp)])
            return _

        lax.fori_loop(0, nch, chunk, 0)

    return k(table, idx)


def _segmax(feat, cl, *, nreal, spw, chs, kb=128):
    del nreal
    npts = cl.shape[0]
    nchunk = npts // chs
    nseg_pad = NC * NS * spw
    ndrain = chs // kb + 1
    cap = chs + kb + 2 * LANES

    mesh = plsc.VectorSubcoreMesh(core_axis_name="c", subcore_axis_name="s",
                                  num_cores=NC, num_subcores=NS)

    @functools.partial(
        pl.kernel, mesh=mesh,
        out_type=jax.ShapeDtypeStruct((nseg_pad, 128), F32),
        scratch_types=[
            pltpu.VMEM((spw, 128), F32),
            pltpu.VMEM((chs,), I32),
            pltpu.VMEM((cap,), I32),
            pltpu.VMEM((cap,), I32),
            pltpu.VMEM((kb, 128), F32),
            pltpu.SemaphoreType.DMA,
        ],
    )
    def k(feat_hbm, cl_hbm, out_hbm, acc, idb, midx, mseg, rows, sem):
        w = lax.axis_index("s") * NC + lax.axis_index("c")
        lo = w * spw
        hi = lo + spw
        it = _iota16()
        z16 = it.astype(F32) * 0.0
        one16 = it * 0 + 1
        zero16 = it * 0

        def zrow(r, _):
            for c in range(8):
                acc[r, pl.ds(c * LANES, LANES)] = z16
            return _

        lax.fori_loop(0, spw, zrow, 0)

        def zstage(j, _):
            midx[pl.ds(j * LANES, LANES)] = j * LANES + it
            mseg[pl.ds(j * LANES, LANES)] = zero16
            return _

        lax.fori_loop(0, cap // LANES, zstage, 0)

        def accum_full(head):
            def grp(b, _):
                base = pl.multiple_of(head + b * LANES, LANES)
                segv = mseg[pl.ds(base, LANES)]
                for t in range(LANES):
                    s = segv[t]
                    bt = b * LANES + t
                    for c in range(8):
                        sl = pl.ds(c * LANES, LANES)
                        acc[s, sl] = jnp.maximum(acc[s, sl], rows[bt, sl])
                return _

            lax.fori_loop(0, kb // LANES, grp, 0)

        def drain_one(st):
            head, off = st
            hh = pl.multiple_of(head, kb)
            pltpu.async_copy(feat_hbm.at[midx.at[pl.ds(hh, kb)]],
                             rows, sem).wait()
            accum_full(hh)
            return (head + kb, off)

        def chunk(g, off):
            pltpu.sync_copy(cl_hbm.at[pl.ds(pl.multiple_of(g * chs, chs),
                                            chs)], idb)

            def step(j, o):
                v = idb[pl.ds(pl.multiple_of(j * LANES, LANES), LANES)]
                d = v - lo
                cs = ((d | (spw - 1 - d)) >> 31) + 1
                for sft in (1, 2, 4, 8):
                    km = ((it - sft) >> 31) + 1
                    cs = cs + km * _lane_bcast(cs, (it - sft) & 15)
                cnt = cs[15]

                def append(o2):
                    pos = zero16
                    for sft in (8, 4, 2, 1):
                        nt = pos + sft
                        vv = _lane_bcast(cs, nt - 1)
                        neg = (vv - it - 1) >> 31
                        pos = pos - neg * (nt - pos)
                    pid = g * chs + j * LANES + it
                    cpid = _lane_bcast(pid, pos)
                    cseg = _lane_bcast(d, pos)
                    r = o2 & (LANES - 1)
                    a0 = pl.multiple_of(o2 & ~(LANES - 1), LANES)
                    a1 = pl.multiple_of(a0 + LANES, LANES)
                    km = (it - r) >> 31
                    rl = (it - r) & 15
                    oldp = midx[pl.ds(a0, LANES)]
                    olds = mseg[pl.ds(a0, LANES)]
                    midx[pl.ds(a0, LANES)] = (oldp & km) | (
                        _lane_bcast(cpid, rl) & ~km)
                    mseg[pl.ds(a0, LANES)] = (olds & km) | (
                        _lane_bcast(cseg, rl) & ~km)
                    rh = (it + LANES - r) & 15
                    midx[pl.ds(a1, LANES)] = _lane_bcast(cpid, rh)
                    mseg[pl.ds(a1, LANES)] = _lane_bcast(cseg, rh)
                    return o2 + cnt

                return lax.cond(cnt > 0, append, lambda o2: o2, o)

            off = lax.fori_loop(0, chs // LANES, step, off)

            st = (0, off)
            for _ in range(ndrain):
                st = lax.cond(st[1] - st[0] >= kb, drain_one, lambda s: s, st)
            head, off = st
            hh = pl.multiple_of(head, kb)

            def mv(t, _):
                src = pl.ds(pl.multiple_of(hh + t * LANES, LANES), LANES)
                dst = pl.ds(t * LANES, LANES)
                midx[dst] = midx[src]
                mseg[dst] = mseg[src]
                return _

            lax.fori_loop(0, kb // LANES, mv, 0)
            return off - head

        off = lax.fori_loop(0, nchunk, chunk, 0)

        @pl.when(off > 0)
        def _():
            pltpu.async_copy(feat_hbm.at[midx.at[pl.ds(0, kb)]],
                             rows, sem).wait()

            def grp(b, _):
                base = pl.multiple_of(b * LANES, LANES)
                segv = mseg[pl.ds(base, LANES)]
                for t in range(LANES):
                    s = jnp.clip(segv[t], 0, spw - 1)
                    bt = b * LANES + t
                    actf = (-((bt - off) >> 31)).astype(F32)
                    for c in range(8):
                        sl = pl.ds(c * LANES, LANES)
                        oldv = acc[s, sl]
                        upd = jnp.maximum(oldv, rows[bt, sl])
                        acc[s, sl] = oldv + (upd - oldv) * actf
                return _

            lax.fori_loop(0, kb // LANES, grp, 0)

        pltpu.sync_copy(acc, out_hbm.at[pl.ds(pl.multiple_of(w * spw, spw),
                                              spw)])

    return k(feat, cl)


def _dot(a, b):
    return jax.lax.dot_general(a, b, (((1,), (0,)), ((), ())),
                               precision=_HIGHEST, preferred_element_type=F32)


def _mlp_pool_tc(x, delta4, w1a, w1b4, b1, w2, b2, *, blk=1024):
    n = x.shape[0]
    grid = pl.cdiv(n, blk)

    def body(x_ref, d_ref, w1a_ref, w1b_ref, b1_ref, w2_ref, b2_ref, o_ref):
        t = _dot(x_ref[...], w1a_ref[...])
        d = d_ref[...]
        wb = w1b_ref[...]
        for kk in range(4):
            t = t + d[:, kk:kk + 1] * wb[kk:kk + 1, :]
        t = jnp.maximum(t + b1_ref[...], 0.0)
        t = jnp.maximum(_dot(t, w2_ref[...]) + b2_ref[...], 0.0)
        o_ref[...] = t

    return pl.pallas_call(
        body,
        grid=(grid,),
        in_specs=[
            pl.BlockSpec((blk, 128), lambda i: (i, 0)),
            pl.BlockSpec((blk, 4), lambda i: (i, 0)),
            pl.BlockSpec((128, 128), lambda i: (0, 0)),
            pl.BlockSpec((4, 128), lambda i: (0, 0)),
            pl.BlockSpec((1, 128), lambda i: (0, 0)),
            pl.BlockSpec((128, 128), lambda i: (0, 0)),
            pl.BlockSpec((1, 128), lambda i: (0, 0)),
        ],
        out_specs=pl.BlockSpec((blk, 128), lambda i: (i, 0)),
        out_shape=jax.ShapeDtypeStruct((n, 128), F32),
    )(x, delta4, w1a, w1b4, b1, w2, b2)


def _mlp_up_tc(a, b, wa, wb, b1, w2, b2, w3, b3, *, blk=1024):
    n = a.shape[0]
    grid = pl.cdiv(n, blk)

    def body(a_ref, b_ref, wa_ref, wb_ref, b1_ref, w2_ref, b2_ref,
             w3_ref, b3_ref, o_ref):
        t = _dot(a_ref[...], wa_ref[...]) + _dot(b_ref[...], wb_ref[...])
        t = jnp.maximum(t + b1_ref[...], 0.0)
        t = jnp.maximum(_dot(t, w2_ref[...]) + b2_ref[...], 0.0)
        o_ref[...] = _dot(t, w3_ref[...]) + b3_ref[...]

    full = lambda i: (0, 0)
    return pl.pallas_call(
        body,
        grid=(grid,),
        in_specs=[
            pl.BlockSpec((blk, 128), lambda i: (i, 0)),
            pl.BlockSpec((blk, 128), lambda i: (i, 0)),
            pl.BlockSpec((128, 128), full),
            pl.BlockSpec((128, 128), full),
            pl.BlockSpec((1, 128), full),
            pl.BlockSpec((128, 128), full),
            pl.BlockSpec((1, 128), full),
            pl.BlockSpec((128, 128), full),
            pl.BlockSpec((1, 128), full),
        ],
        out_specs=pl.BlockSpec((blk, 128), lambda i: (i, 0)),
        out_shape=jax.ShapeDtypeStruct((n, 128), F32),
    )(a, b, wa, wb, b1, w2, b2, w3, b3)


def kernel(x, pos, params, cluster0, cluster1):
    n0 = x.shape[0]
    n1 = cluster1.shape[0]
    n2 = 6250 if n1 == 25000 else int(jnp.max(cluster1)) + 1

    cluster0 = cluster0.astype(I32)
    cluster1 = cluster1.astype(I32)

    np0 = 102400
    np1 = 25600
    spw0 = 784
    nseg0 = NC * NS * spw0
    spw1 = 200
    nseg1 = NC * NS * spw1

    pad0 = np0 - n0
    pad1 = np1 - n1

    cl0d = jnp.concatenate([cluster0, jnp.full((pad0,), nseg0, I32)])
    cl0g = jnp.concatenate([cluster0, (jnp.arange(pad0, dtype=I32) % n1)])
    cl1d = jnp.concatenate([cluster1, jnp.full((pad1,), nseg1, I32)])
    cl1g = jnp.concatenate([cluster1, (jnp.arange(pad1, dtype=I32) % n2)])
    cl0a = jnp.concatenate([cluster0, jnp.full((pad0,), n1, I32)])
    cl1a = jnp.concatenate([cluster1, jnp.full((pad1,), n2, I32)])

    pospad = jnp.concatenate(
        [pos, jnp.ones((n0, 1), F32)], axis=1)
    pospadp = jnp.concatenate([pospad, jnp.zeros((pad0, 4), F32)])

    pospadp_flat = pospadp.reshape(-1)

    sums0 = _segadd4(pospadp_flat, cl0a, nseg_pad=nseg0, ch=800)
    p0n_flat = _norm4(sums0)
    d0_flat = _gather_sub4(p0n_flat, cl0g, pospadp_flat, cp=800, nch=4)
    delta0 = d0_flat.reshape(np0, 4)[:n0]

    pw = params["pool0"]
    feat0 = _mlp_pool_tc(
        x, delta0,
        pw["W"][0][:128], jnp.concatenate([pw["W"][0][128:],
                                           jnp.zeros((1, 128), F32)]),
        pw["b"][0].reshape(1, 128), pw["W"][1], pw["b"][1].reshape(1, 128))

    h1p = _segmax(feat0, cl0d, nreal=n0, spw=spw0, chs=512)
    h1 = h1p[:n1]

    p0np_flat = jnp.concatenate(
        [p0n_flat, jnp.zeros(((np1 - nseg0) * 4,), F32)])
    sums1 = _segadd4(p0np_flat, cl1a, nseg_pad=nseg0, ch=800)
    p1n_flat = _norm4(sums1)
    d1_flat = _gather_sub4(p1n_flat, cl1g, p0np_flat, cp=800, nch=1)
    delta1 = d1_flat.reshape(np1, 4)[:n1]

    pw = params["pool1"]
    feat1 = _mlp_pool_tc(
        h1, delta1,
        pw["W"][0][:128], jnp.concatenate([pw["W"][0][128:],
                                           jnp.zeros((1, 128), F32)]),
        pw["b"][0].reshape(1, 128), pw["W"][1], pw["b"][1].reshape(1, 128))

    h2p = _segmax(feat1, cl1d, nreal=n1, spw=spw1, chs=512)

    hup = _gather_rows(h2p, cl1g, cp=800, nch=1)[:n1]
    uw = params["up0"]
    g = _mlp_up_tc(hup, h1,
                   uw["W"][0][:128], uw["W"][0][128:],
                   uw["b"][0].reshape(1, 128), uw["W"][1],
                   uw["b"][1].reshape(1, 128), uw["W"][2],
                   uw["b"][2].reshape(1, 128))

    gup = _gather_rows(g, cl0g, cp=800, nch=4)[:n0]
    uw = params["up1"]
    out = _mlp_up_tc(gup, x,
                     uw["W"][0][:128], uw["W"][0][128:],
                     uw["b"][0].reshape(1, 128), uw["W"][1],
                     uw["b"][1].reshape(1, 128), uw["W"][2],
                     uw["b"][2].reshape(1, 128))
    return out


# speedup vs baseline: 1.0342x; 1.0342x over previous
import functools

import jax
import jax.numpy as jnp
from jax import lax
from jax.experimental import pallas as pl
from jax.experimental.pallas import tpu as pltpu
from jax.experimental.pallas import tpu_sc as plsc

F32 = jnp.float32
I32 = jnp.int32

NC = 2
NS = 16
LANES = 16

_HIGHEST = lax.Precision.HIGHEST


def _iota16():
    return lax.iota(I32, LANES)


def _lane_bcast(v, idx_const):
    dnums = lax.GatherDimensionNumbers(
        offset_dims=(), collapsed_slice_dims=(0,), start_index_map=(0,))
    return lax.gather(v, idx_const[:, None], dnums, slice_sizes=(1,),
                      mode=lax.GatherScatterMode.PROMISE_IN_BOUNDS)


def _segadd4(updf, idx, *, nseg_pad, ch):
    npts = idx.shape[0]
    tpw = npts // (NC * NS)
    nch = tpw // ch
    se = nseg_pad * 4 // NS
    ce = ch * 4

    mesh = plsc.VectorSubcoreMesh(core_axis_name="c", subcore_axis_name="s",
                                  num_cores=NC, num_subcores=NS)

    @functools.partial(
        pl.kernel, mesh=mesh,
        out_type=jax.ShapeDtypeStruct((NC * nseg_pad * 4,), F32),
        scratch_types=[
            pltpu.VMEM_SHARED((nseg_pad * 4,), F32),
            pltpu.VMEM((ch + LANES,), I32),
            pltpu.VMEM((ce,), I32),
            pltpu.VMEM((ce,), F32),
            pltpu.VMEM((se,), F32),
        ],
    )
    def k(upd_hbm, idx_hbm, out_hbm, acc, idxb, eidx, updb, zbuf):
        c = lax.axis_index("c")
        t = lax.axis_index("s")
        w = t * NC + c
        it = _iota16()
        f4 = it & 3
        z16 = it.astype(F32) * 0.0

        def zv(i, _):
            zbuf[pl.ds(pl.multiple_of(i * LANES, LANES), LANES)] = z16
            return _

        lax.fori_loop(0, se // LANES, zv, 0)
        toff = pl.multiple_of(t * se, se)
        pltpu.sync_copy(zbuf, acc.at[pl.ds(toff, se)])
        plsc.subcore_barrier()

        def chunk(g, _):
            off = pl.multiple_of(w * tpw + g * ch, ch)
            pltpu.sync_copy(idx_hbm.at[pl.ds(off, ch)], idxb.at[pl.ds(0, ch)])
            eoff = pl.multiple_of(off * 4, ce)
            pltpu.sync_copy(upd_hbm.at[pl.ds(eoff, ce)], updb)

            def build16(j, _2):
                base16 = idxb[pl.ds(pl.multiple_of(j * 4, 4), LANES)]
                ids4 = _lane_bcast(base16, it >> 2)
                eidx[pl.ds(pl.multiple_of(j * LANES, LANES), LANES)] = (
                    ids4 * 4 + f4)
                return _2

            lax.fori_loop(0, ce // LANES, build16, 0)
            pltpu.sync_copy(updb, acc.at[eidx], add=True)
            return _

        lax.fori_loop(0, nch, chunk, 0)
        plsc.subcore_barrier()
        ooff = pl.multiple_of(c * nseg_pad * 4 + t * se, se)
        pltpu.sync_copy(acc.at[pl.ds(toff, se)], out_hbm.at[pl.ds(ooff, se)])

    return k(updf, idx)


def _norm4(sums2_flat):
    ne2 = sums2_flat.shape[0]
    ne = ne2 // NC
    per_w = ne // (NC * NS)
    nvec = per_w // LANES

    mesh = plsc.VectorSubcoreMesh(core_axis_name="c", subcore_axis_name="s",
                                  num_cores=NC, num_subcores=NS)

    @functools.partial(
        pl.kernel, mesh=mesh,
        out_type=jax.ShapeDtypeStruct((ne,), F32),
        scratch_types=[pltpu.VMEM((per_w,), F32), pltpu.VMEM((per_w,), F32)],
    )
    def k(sums_hbm, out_hbm, buf, buf2):
        w = lax.axis_index("s") * NC + lax.axis_index("c")
        toff = pl.multiple_of(w * per_w, per_w)
        toff2 = pl.multiple_of(ne + w * per_w, per_w)
        pltpu.sync_copy(sums_hbm.at[pl.ds(toff, per_w)], buf)
        pltpu.sync_copy(sums_hbm.at[pl.ds(toff2, per_w)], buf2)
        it = _iota16()
        p3 = it | 3

        def body(i, _):
            s = pl.ds(i * LANES, LANES)
            g = buf[s] + buf2[s]
            cn = jnp.maximum(_lane_bcast(g, p3), 1.0)
            r = g / cn
            mf = (((it & 3) - 3) >> 31).astype(F32) + 1.0
            buf[s] = r * (1.0 - mf) + mf
            return _

        lax.fori_loop(0, nvec, body, 0)
        pltpu.sync_copy(buf, out_hbm.at[pl.ds(toff, per_w)])

    return k(sums2_flat)


def _gather_sub4(table_flat, idx, base_flat, *, cp, nch):
    npts = idx.shape[0]
    assert npts == NC * NS * nch * cp
    ce = cp * 4

    mesh = plsc.VectorSubcoreMesh(core_axis_name="c", subcore_axis_name="s",
                                  num_cores=NC, num_subcores=NS)

    @functools.partial(
        pl.kernel, mesh=mesh,
        out_type=jax.ShapeDtypeStruct((npts * 4,), F32),
        scratch_types=[
            pltpu.VMEM((cp + LANES,), I32),
            pltpu.VMEM((ce,), I32),
            pltpu.VMEM((ce,), F32),
            pltpu.VMEM((ce,), F32),
            pltpu.SemaphoreType.DMA,
        ],
    )
    def k(tab_hbm, idx_hbm, base_hbm, out_hbm, idxb, eidx, g, b, sem):
        w = lax.axis_index("s") * NC + lax.axis_index("c")
        it = _iota16()
        f4 = it & 3

        def chunk(ci, _):
            off = pl.multiple_of((w * nch + ci) * cp, cp)
            pltpu.sync_copy(idx_hbm.at[pl.ds(off, cp)], idxb.at[pl.ds(0, cp)])

            def build16(j, _):
                base16 = idxb[pl.ds(j * 4, LANES)]
                ids4 = _lane_bcast(base16, it >> 2)
                eidx[pl.ds(j * LANES, LANES)] = ids4 * 4 + f4
                return _

            lax.fori_loop(0, ce // LANES, build16, 0)
            pltpu.async_copy(tab_hbm.at[eidx], g, sem).wait()
            eoff = pl.multiple_of(off * 4, ce)
            pltpu.sync_copy(base_hbm.at[pl.ds(eoff, ce)], b)

            def sub(j, _):
                s = pl.ds(j * LANES, LANES)
                b[s] = b[s] - g[s]
                return _

            lax.fori_loop(0, ce // LANES, sub, 0)
            pltpu.sync_copy(b, out_hbm.at[pl.ds(eoff, ce)])
            return _

        lax.fori_loop(0, nch, chunk, 0)

    return k(table_flat, idx, base_flat)


def _gather_rows(table, idx, *, cp, nch):
    npts = idx.shape[0]
    assert npts == NC * NS * nch * cp
    d = table.shape[1]

    mesh = plsc.VectorSubcoreMesh(core_axis_name="c", subcore_axis_name="s",
                                  num_cores=NC, num_subcores=NS)

    @functools.partial(
        pl.kernel, mesh=mesh,
        out_type=jax.ShapeDtypeStruct((npts, d), F32),
        scratch_types=[
            pltpu.VMEM((cp,), I32),
            pltpu.VMEM((cp, d), F32),
            pltpu.SemaphoreType.DMA,
        ],
    )
    def k(tab_hbm, idx_hbm, out_hbm, idxb, rows, sem):
        w = lax.axis_index("s") * NC + lax.axis_index("c")

        def chunk(ci, _):
            off = pl.multiple_of((w * nch + ci) * cp, cp)
            pltpu.sync_copy(idx_hbm.at[pl.ds(off, cp)], idxb)
            pltpu.async_copy(tab_hbm.at[idxb], rows, sem).wait()
            pltpu.sync_copy(rows, out_hbm.at[pl.ds(off, cp)])
            return _

        lax.fori_loop(0, nch, chunk, 0)

    return k(table, idx)


def _segmax(feat, cl, *, nreal, spw, chs, kb=128):
    del nreal
    npts = cl.shape[0]
    nchunk = npts // chs
    nseg_pad = NC * NS * spw
    ndrain = chs // kb + 1
    cap = chs + kb + 2 * LANES

    mesh = plsc.VectorSubcoreMesh(core_axis_name="c", subcore_axis_name="s",
                                  num_cores=NC, num_subcores=NS)

    @functools.partial(
        pl.kernel, mesh=mesh,
        out_type=jax.ShapeDtypeStruct((nseg_pad, 128), F32),
        scratch_types=[
            pltpu.VMEM((spw, 128), F32),
            pltpu.VMEM((chs,), I32),
            pltpu.VMEM((cap,), I32),
            pltpu.VMEM((cap,), I32),
            pltpu.VMEM((kb, 128), F32),
            pltpu.SemaphoreType.DMA,
        ],
    )
    def k(feat_hbm, cl_hbm, out_hbm, acc, idb, midx, mseg, rows, sem):
        w = lax.axis_index("s") * NC + lax.axis_index("c")
        lo = w * spw
        hi = lo + spw
        it = _iota16()
        z16 = it.astype(F32) * 0.0
        one16 = it * 0 + 1
        zero16 = it * 0

        def zrow(r, _):
            for c in range(8):
                acc[r, pl.ds(c * LANES, LANES)] = z16
            return _

        lax.fori_loop(0, spw, zrow, 0)

        def zstage(j, _):
            midx[pl.ds(j * LANES, LANES)] = j * LANES + it
            mseg[pl.ds(j * LANES, LANES)] = zero16
            return _

        lax.fori_loop(0, cap // LANES, zstage, 0)

        def accum_full(head):
            def grp(b, _):
                base = pl.multiple_of(head + b * LANES, LANES)
                segv = mseg[pl.ds(base, LANES)]
                for t in range(LANES):
                    s = segv[t]
                    bt = b * LANES + t
                    for c in range(8):
                        sl = pl.ds(c * LANES, LANES)
                        acc[s, sl] = jnp.maximum(acc[s, sl], rows[bt, sl])
                return _

            lax.fori_loop(0, kb // LANES, grp, 0)

        def drain_one(st):
            head, off = st
            hh = pl.multiple_of(head, kb)
            pltpu.async_copy(feat_hbm.at[midx.at[pl.ds(hh, kb)]],
                             rows, sem).wait()
            accum_full(hh)
            return (head + kb, off)

        def chunk(g, off):
            pltpu.sync_copy(cl_hbm.at[pl.ds(pl.multiple_of(g * chs, chs),
                                            chs)], idb)

            def step(j, o):
                v = idb[pl.ds(pl.multiple_of(j * LANES, LANES), LANES)]
                d = v - lo
                cs = ((d | (spw - 1 - d)) >> 31) + 1
                for sft in (1, 2, 4, 8):
                    km = ((it - sft) >> 31) + 1
                    cs = cs + km * _lane_bcast(cs, (it - sft) & 15)
                cnt = cs[15]

                def append(o2):
                    pos = zero16
                    for sft in (8, 4, 2, 1):
                        nt = pos + sft
                        vv = _lane_bcast(cs, nt - 1)
                        neg = (vv - it - 1) >> 31
                        pos = pos - neg * (nt - pos)
                    pid = g * chs + j * LANES + it
                    cpid = _lane_bcast(pid, pos)
                    cseg = _lane_bcast(d, pos)
                    r = o2 & (LANES - 1)
                    a0 = pl.multiple_of(o2 & ~(LANES - 1), LANES)
                    a1 = pl.multiple_of(a0 + LANES, LANES)
                    km = (it - r) >> 31
                    rl = (it - r) & 15
                    oldp = midx[pl.ds(a0, LANES)]
                    olds = mseg[pl.ds(a0, LANES)]
                    midx[pl.ds(a0, LANES)] = (oldp & km) | (
                        _lane_bcast(cpid, rl) & ~km)
                    mseg[pl.ds(a0, LANES)] = (olds & km) | (
                        _lane_bcast(cseg, rl) & ~km)
                    rh = (it + LANES - r) & 15
                    midx[pl.ds(a1, LANES)] = _lane_bcast(cpid, rh)
                    mseg[pl.ds(a1, LANES)] = _lane_bcast(cseg, rh)
                    return o2 + cnt

                return lax.cond(cnt > 0, append, lambda o2: o2, o)

            off = lax.fori_loop(0, chs // LANES, step, off)

            st = (0, off)
            for _ in range(ndrain):
                st = lax.cond(st[1] - st[0] >= kb, drain_one, lambda s: s, st)
            head, off = st
            hh = pl.multiple_of(head, kb)

            def mv(t, _):
                src = pl.ds(pl.multiple_of(hh + t * LANES, LANES), LANES)
                dst = pl.ds(t * LANES, LANES)
                midx[dst] = midx[src]
                mseg[dst] = mseg[src]
                return _

            lax.fori_loop(0, kb // LANES, mv, 0)
            return off - head

        off = lax.fori_loop(0, nchunk, chunk, 0)

        @pl.when(off > 0)
        def _():
            pltpu.async_copy(feat_hbm.at[midx.at[pl.ds(0, kb)]],
                             rows, sem).wait()

            def grp(b, _):
                base = pl.multiple_of(b * LANES, LANES)
                segv = mseg[pl.ds(base, LANES)]
                for t in range(LANES):
                    s = jnp.clip(segv[t], 0, spw - 1)
                    bt = b * LANES + t
                    actf = (-((bt - off) >> 31)).astype(F32)
                    for c in range(8):
                        sl = pl.ds(c * LANES, LANES)
                        oldv = acc[s, sl]
                        upd = jnp.maximum(oldv, rows[bt, sl])
                        acc[s, sl] = oldv + (upd - oldv) * actf
                return _

            lax.fori_loop(0, kb // LANES, grp, 0)

        pltpu.sync_copy(acc, out_hbm.at[pl.ds(pl.multiple_of(w * spw, spw),
                                              spw)])

    return k(feat, cl)


def _dot(a, b):
    return jax.lax.dot_general(a, b, (((1,), (0,)), ((), ())),
                               precision=_HIGHEST, preferred_element_type=F32)


def _mlp_pool_tc(x, delta4, w1a, w1b4, b1, w2, b2, *, blk=1024):
    n = x.shape[0]
    grid = pl.cdiv(n, blk)

    def body(x_ref, d_ref, w1a_ref, w1b_ref, b1_ref, w2_ref, b2_ref, o_ref):
        t = _dot(x_ref[...], w1a_ref[...])
        d = d_ref[...]
        wb = w1b_ref[...]
        for kk in range(4):
            t = t + d[:, kk:kk + 1] * wb[kk:kk + 1, :]
        t = jnp.maximum(t + b1_ref[...], 0.0)
        t = jnp.maximum(_dot(t, w2_ref[...]) + b2_ref[...], 0.0)
        o_ref[...] = t

    return pl.pallas_call(
        body,
        grid=(grid,),
        in_specs=[
            pl.BlockSpec((blk, 128), lambda i: (i, 0)),
            pl.BlockSpec((blk, 4), lambda i: (i, 0)),
            pl.BlockSpec((128, 128), lambda i: (0, 0)),
            pl.BlockSpec((4, 128), lambda i: (0, 0)),
            pl.BlockSpec((1, 128), lambda i: (0, 0)),
            pl.BlockSpec((128, 128), lambda i: (0, 0)),
            pl.BlockSpec((1, 128), lambda i: (0, 0)),
        ],
        out_specs=pl.BlockSpec((blk, 128), lambda i: (i, 0)),
        out_shape=jax.ShapeDtypeStruct((n, 128), F32),
    )(x, delta4, w1a, w1b4, b1, w2, b2)


def _mlp_up_tc(a, b, wa, wb, b1, w2, b2, w3, b3, *, blk=1024):
    n = a.shape[0]
    grid = pl.cdiv(n, blk)

    def body(a_ref, b_ref, wa_ref, wb_ref, b1_ref, w2_ref, b2_ref,
             w3_ref, b3_ref, o_ref):
        t = _dot(a_ref[...], wa_ref[...]) + _dot(b_ref[...], wb_ref[...])
        t = jnp.maximum(t + b1_ref[...], 0.0)
        t = jnp.maximum(_dot(t, w2_ref[...]) + b2_ref[...], 0.0)
        o_ref[...] = _dot(t, w3_ref[...]) + b3_ref[...]

    full = lambda i: (0, 0)
    return pl.pallas_call(
        body,
        grid=(grid,),
        in_specs=[
            pl.BlockSpec((blk, 128), lambda i: (i, 0)),
            pl.BlockSpec((blk, 128), lambda i: (i, 0)),
            pl.BlockSpec((128, 128), full),
            pl.BlockSpec((128, 128), full),
            pl.BlockSpec((1, 128), full),
            pl.BlockSpec((128, 128), full),
            pl.BlockSpec((1, 128), full),
            pl.BlockSpec((128, 128), full),
            pl.BlockSpec((1, 128), full),
        ],
        out_specs=pl.BlockSpec((blk, 128), lambda i: (i, 0)),
        out_shape=jax.ShapeDtypeStruct((n, 128), F32),
    )(a, b, wa, wb, b1, w2, b2, w3, b3)


def kernel(x, pos, params, cluster0, cluster1):
    n0 = x.shape[0]
    n1 = cluster1.shape[0]
    n2 = 6250 if n1 == 25000 else int(jnp.max(cluster1)) + 1

    cluster0 = cluster0.astype(I32)
    cluster1 = cluster1.astype(I32)

    np0 = 102400
    np1 = 25600
    spw0 = 784
    nseg0 = NC * NS * spw0
    spw1 = 200
    nseg1 = NC * NS * spw1

    pad0 = np0 - n0
    pad1 = np1 - n1

    cl0d = jnp.concatenate([cluster0, jnp.full((pad0,), nseg0, I32)])
    cl0g = jnp.concatenate([cluster0, (jnp.arange(pad0, dtype=I32) % n1)])
    cl1d = jnp.concatenate([cluster1, jnp.full((pad1,), nseg1, I32)])
    cl1g = jnp.concatenate([cluster1, (jnp.arange(pad1, dtype=I32) % n2)])
    cl0a = jnp.concatenate([cluster0, jnp.full((pad0,), n1, I32)])
    cl1a = jnp.concatenate([cluster1, jnp.full((pad1,), n2, I32)])

    pospad = jnp.concatenate(
        [pos, jnp.ones((n0, 1), F32)], axis=1)
    pospadp = jnp.concatenate([pospad, jnp.zeros((pad0, 4), F32)])

    pospadp_flat = pospadp.reshape(-1)

    sums0 = _segadd4(pospadp_flat, cl0a, nseg_pad=nseg0, ch=800)
    p0n_flat = _norm4(sums0)
    d0_flat = _gather_sub4(p0n_flat, cl0g, pospadp_flat, cp=800, nch=4)
    delta0 = d0_flat.reshape(np0, 4)[:n0]

    pw = params["pool0"]
    feat0 = _mlp_pool_tc(
        x, delta0,
        pw["W"][0][:128], jnp.concatenate([pw["W"][0][128:],
                                           jnp.zeros((1, 128), F32)]),
        pw["b"][0].reshape(1, 128), pw["W"][1], pw["b"][1].reshape(1, 128))

    h1p = _segmax(feat0, cl0d, nreal=n0, spw=spw0, chs=1024)
    h1 = h1p[:n1]

    p0np_flat = jnp.concatenate(
        [p0n_flat, jnp.zeros(((np1 - nseg0) * 4,), F32)])
    sums1 = _segadd4(p0np_flat, cl1a, nseg_pad=nseg0, ch=800)
    p1n_flat = _norm4(sums1)
    d1_flat = _gather_sub4(p1n_flat, cl1g, p0np_flat, cp=800, nch=1)
    delta1 = d1_flat.reshape(np1, 4)[:n1]

    pw = params["pool1"]
    feat1 = _mlp_pool_tc(
        h1, delta1,
        pw["W"][0][:128], jnp.concatenate([pw["W"][0][128:],
                                           jnp.zeros((1, 128), F32)]),
        pw["b"][0].reshape(1, 128), pw["W"][1], pw["b"][1].reshape(1, 128))

    h2p = _segmax(feat1, cl1d, nreal=n1, spw=spw1, chs=800)

    hup = _gather_rows(h2p, cl1g, cp=800, nch=1)[:n1]
    uw = params["up0"]
    g = _mlp_up_tc(hup, h1,
                   uw["W"][0][:128], uw["W"][0][128:],
                   uw["b"][0].reshape(1, 128), uw["W"][1],
                   uw["b"][1].reshape(1, 128), uw["W"][2],
                   uw["b"][2].reshape(1, 128))

    gup = _gather_rows(g, cl0g, cp=800, nch=4)[:n0]
    uw = params["up1"]
    out = _mlp_up_tc(gup, x,
                     uw["W"][0][:128], uw["W"][0][128:],
                     uw["b"][0].reshape(1, 128), uw["W"][1],
                     uw["b"][1].reshape(1, 128), uw["W"][2],
                     uw["b"][2].reshape(1, 128))
    return out
